# baseline (speedup 1.0000x reference)
import functools

import jax
import jax.numpy as jnp
from jax import lax
from jax.experimental import pallas as pl
from jax.experimental.pallas import tpu as pltpu
from jax.experimental.pallas import tpu_sc as plsc

S, B, D = 26, 4096, 128
W = D + 1

NC, NS = 2, 16
NW = NC * NS
CB = B // NW
TB = 16
NBLK = CB // TB

_mesh = plsc.VectorSubcoreMesh(
    core_axis_name="c", subcore_axis_name="s", num_cores=NC, num_subcores=NS
)


@functools.partial(
    pl.kernel,
    out_type=jax.ShapeDtypeStruct((B, S, W), jnp.float32),
    mesh=_mesh,
    scratch_types=[
        pltpu.VMEM((S, CB), jnp.int32),
        pltpu.VMEM((TB, S, W), jnp.float32),
        pltpu.SemaphoreType.DMA,
    ],
    compiler_params=pltpu.CompilerParams(
        use_tc_tiling_on_sc=False, needs_layout_passes=False
    ),
)
def _fuse(x_hbm, tasks_hbm, out_hbm, ti, stg, sem):
    wid = lax.axis_index("s") * NC + lax.axis_index("c")
    base = wid * CB
    lane = lax.iota(jnp.int32, 16)

    pltpu.sync_copy(tasks_hbm.at[:, pl.ds(base, CB)], ti)

    col_d = jnp.full((16,), D, jnp.int32)

    for blk in range(NBLK):
        b0 = base + blk * TB

        def _fire(s_i, _):
            pltpu.async_copy(
                x_hbm.at[s_i, pl.ds(b0, TB), :],
                stg.at[:, s_i, pl.ds(0, D)],
                sem,
            )
            return 0

        lax.fori_loop(0, S, _fire, 0)

        def _drain(s_i, _):
            pltpu.make_async_copy(
                x_hbm.at[s_i, pl.ds(b0, TB), :],
                stg.at[:, s_i, pl.ds(0, D)],
                sem,
            ).wait()
            return 0

        lax.fori_loop(0, S, _drain, 0)

        def _taskcol(s_i, _):
            t = ti[s_i, pl.ds(blk * TB, TB)]
            vals = (1 - 2 * t).astype(jnp.float32)
            plsc.store_scatter(
                stg,
                [lane, jnp.full((16,), s_i, jnp.int32), col_d],
                vals,
            )
            return 0

        lax.fori_loop(0, S, _taskcol, 0)

        pltpu.sync_copy(stg, out_hbm.at[pl.ds(b0, TB)])


def kernel(x, tasks):
    return _fuse(x, tasks).reshape(B, S * W)


# speedup vs baseline: 1.0237x; 1.0237x over previous
import functools

import jax
import jax.numpy as jnp
from jax import lax
from jax.experimental import pallas as pl
from jax.experimental.pallas import tpu as pltpu
from jax.experimental.pallas import tpu_sc as plsc

S, B, D = 26, 4096, 128
W = D + 1

NC, NS = 2, 16
NW = NC * NS
CB = B // NW
TB = 16
NBLK = CB // TB

_mesh = plsc.VectorSubcoreMesh(
    core_axis_name="c", subcore_axis_name="s", num_cores=NC, num_subcores=NS
)


@functools.partial(
    pl.kernel,
    out_type=jax.ShapeDtypeStruct((B, S, W), jnp.float32),
    mesh=_mesh,
    scratch_types=[
        pltpu.VMEM((S, CB), jnp.int32),
        pltpu.VMEM((2, TB, S, W), jnp.float32),
        pltpu.SemaphoreType.DMA,
        pltpu.SemaphoreType.DMA,
        pltpu.SemaphoreType.DMA,
    ],
    compiler_params=pltpu.CompilerParams(
        use_tc_tiling_on_sc=False, needs_layout_passes=False
    ),
)
def _fuse(x_hbm, tasks_hbm, out_hbm, ti, stg, sem_in, sem_o0, sem_o1):
    wid = lax.axis_index("s") * NC + lax.axis_index("c")
    base = wid * CB
    lane = lax.iota(jnp.int32, 16)
    col_d = jnp.full((16,), D, jnp.int32)
    sem_out = (sem_o0, sem_o1)

    pltpu.sync_copy(tasks_hbm.at[:, pl.ds(base, CB)], ti)

    def fire_in(blk, buf):
        b0 = base + blk * TB

        def _fire(s_i, _):
            pltpu.async_copy(
                x_hbm.at[s_i, pl.ds(b0, TB), :],
                stg.at[buf, :, s_i, pl.ds(0, D)],
                sem_in,
            )
            return 0

        lax.fori_loop(0, S, _fire, 0)

    def drain_in(buf):
        pltpu.make_async_copy(
            x_hbm.at[pl.ds(0, TB), pl.ds(0, S), :],
            stg.at[buf, :, :, pl.ds(0, D)],
            sem_in,
        ).wait()

    def taskcol(blk, buf):
        def _one(s_i, _):
            t = ti[s_i, pl.ds(blk * TB, TB)]
            vals = (1 - 2 * t).astype(jnp.float32)
            plsc.store_scatter(
                stg.at[buf],
                [lane, jnp.full((16,), s_i, jnp.int32), col_d],
                vals,
            )
            return 0

        lax.fori_loop(0, S, _one, 0)

    def out_copy(blk, buf):
        return pltpu.make_async_copy(
            stg.at[buf], out_hbm.at[pl.ds(base + blk * TB, TB)], sem_out[buf]
        )

    fire_in(0, 0)
    for blk in range(NBLK):
        buf = blk % 2
        drain_in(buf)
        taskcol(blk, buf)
        out_copy(blk, buf).start()
        if blk + 1 < NBLK:
            nxt = (blk + 1) % 2
            if blk >= 1:
                out_copy(blk - 1, nxt).wait()
            fire_in(blk + 1, nxt)
    out_copy(NBLK - 2, (NBLK - 2) % 2).wait()
    out_copy(NBLK - 1, (NBLK - 1) % 2).wait()


def kernel(x, tasks):
    return _fuse(x, tasks).reshape(B, S * W)


# speedup vs baseline: 1.0243x; 1.0006x over previous
import functools

import jax
import jax.numpy as jnp
from jax import lax
from jax.experimental import pallas as pl
from jax.experimental.pallas import tpu as pltpu
from jax.experimental.pallas import tpu_sc as plsc

S, B, D = 26, 4096, 128
W = D + 1

NC, NS = 2, 16
NW = NC * NS
CB = B // NW
TB = 32
NBLK = CB // TB

_mesh = plsc.VectorSubcoreMesh(
    core_axis_name="c", subcore_axis_name="s", num_cores=NC, num_subcores=NS
)


@functools.partial(
    pl.kernel,
    out_type=jax.ShapeDtypeStruct((B, S, W), jnp.float32),
    mesh=_mesh,
    scratch_types=[
        pltpu.VMEM((S, CB), jnp.int32),
        pltpu.VMEM((1, TB, S, W), jnp.float32),
        pltpu.SemaphoreType.DMA,
        pltpu.SemaphoreType.DMA,
        pltpu.SemaphoreType.DMA,
    ],
    compiler_params=pltpu.CompilerParams(
        use_tc_tiling_on_sc=False, needs_layout_passes=False
    ),
)
def _fuse(x_hbm, tasks_hbm, out_hbm, ti, stg, sem_in, sem_o0, sem_o1):
    wid = lax.axis_index("s") * NC + lax.axis_index("c")
    base = wid * CB
    lane = lax.iota(jnp.int32, 16)
    col_d = jnp.full((16,), D, jnp.int32)
    sem_out = (sem_o0, sem_o1)

    pltpu.sync_copy(tasks_hbm.at[:, pl.ds(base, CB)], ti)

    def fire_in(blk, buf):
        b0 = base + blk * TB

        def _fire(s_i, _):
            pltpu.async_copy(
                x_hbm.at[s_i, pl.ds(b0, TB), :],
                stg.at[buf, :, s_i, pl.ds(0, D)],
                sem_in,
            )
            return 0

        lax.fori_loop(0, S, _fire, 0)

    def drain_in(buf):
        pltpu.make_async_copy(
            x_hbm.at[pl.ds(0, TB), pl.ds(0, S), :],
            stg.at[buf, :, :, pl.ds(0, D)],
            sem_in,
        ).wait()

    def taskcol(blk, buf):
        def _one(s_i, _):
            for j in range(TB // 16):
                t = ti[s_i, pl.ds(blk * TB + j * 16, 16)]
                vals = (1 - 2 * t).astype(jnp.float32)
                plsc.store_scatter(
                    stg.at[buf],
                    [lane + j * 16, jnp.full((16,), s_i, jnp.int32), col_d],
                    vals,
                )
            return 0

        lax.fori_loop(0, S, _one, 0)

    def out_copy(blk, buf):
        return pltpu.make_async_copy(
            stg.at[buf], out_hbm.at[pl.ds(base + blk * TB, TB)], sem_out[buf]
        )

    for blk in range(NBLK):
        fire_in(blk, 0)
        drain_in(0)
        taskcol(blk, 0)
        out_copy(blk, 0).start()
        out_copy(blk, 0).wait()


def kernel(x, tasks):
    return _fuse(x, tasks).reshape(B, S * W)


# speedup vs baseline: 6.5789x; 6.4229x over previous
import jax
import jax.numpy as jnp
from jax.experimental import pallas as pl

S, B, D = 26, 4096, 128
W = D + 1
SG = 8
FB = SG * W
BN = 512
GK = -(-S // SG)
GJ = B // BN


def _body(x_ref, t_ref, z_ref):
    for si in range(SG):
        xt = jnp.transpose(x_ref[si], (1, 0))
        z_ref[pl.ds(si * W, D), :] = xt
        tv = (1 - 2 * t_ref[si]).astype(jnp.float32)
        z_ref[pl.ds(si * W + D, 1), :] = tv[None, :]


_call = pl.pallas_call(
    _body,
    grid=(GK, GJ),
    in_specs=[
        pl.BlockSpec((SG, BN, D), lambda k, j: (k, j, 0)),
        pl.BlockSpec((SG, BN), lambda k, j: (k, j)),
    ],
    out_specs=pl.BlockSpec((FB, BN), lambda k, j: (k, j)),
    out_shape=jax.ShapeDtypeStruct((S * W, B), jnp.float32),
)


def kernel(x, tasks):
    return _call(x, tasks).T


# speedup vs baseline: 7.8598x; 1.1947x over previous
import jax
import jax.numpy as jnp
from jax.experimental import pallas as pl

S, B, D = 26, 4096, 128
W = D + 1
SG = 8
FB = SG * W
BN = 1024
GK = -(-S // SG)
GJ = B // BN


def _body(x_ref, t_ref, z_ref):
    for si in range(SG):
        xt = jnp.transpose(x_ref[si], (1, 0))
        z_ref[pl.ds(si * W, D), :] = xt
        tv = (1 - 2 * t_ref[si]).astype(jnp.float32)
        z_ref[pl.ds(si * W + D, 1), :] = tv[None, :]


_call = pl.pallas_call(
    _body,
    grid=(GK, GJ),
    in_specs=[
        pl.BlockSpec((SG, BN, D), lambda k, j: (k, j, 0)),
        pl.BlockSpec((SG, BN), lambda k, j: (k, j)),
    ],
    out_specs=pl.BlockSpec((FB, BN), lambda k, j: (k, j)),
    out_shape=jax.ShapeDtypeStruct((S * W, B), jnp.float32),
)


def kernel(x, tasks):
    return _call(x, tasks).T


# speedup vs baseline: 8.4013x; 1.0689x over previous
import jax
import jax.numpy as jnp
from jax.experimental import pallas as pl

S, B, D = 26, 4096, 128
W = D + 1
SG = 8
FB = SG * W
BN = 2048
GK = -(-S // SG)
GJ = B // BN


def _body(x_ref, t_ref, z_ref):
    for si in range(SG):
        xt = jnp.transpose(x_ref[si], (1, 0))
        z_ref[pl.ds(si * W, D), :] = xt
        tv = (1 - 2 * t_ref[si]).astype(jnp.float32)
        z_ref[pl.ds(si * W + D, 1), :] = tv[None, :]


_call = pl.pallas_call(
    _body,
    grid=(GK, GJ),
    in_specs=[
        pl.BlockSpec((SG, BN, D), lambda k, j: (k, j, 0)),
        pl.BlockSpec((SG, BN), lambda k, j: (k, j)),
    ],
    out_specs=pl.BlockSpec((FB, BN), lambda k, j: (k, j)),
    out_shape=jax.ShapeDtypeStruct((S * W, B), jnp.float32),
)


def kernel(x, tasks):
    return _call(x, tasks).T
